# baseline (speedup 1.0000x reference)
import functools

import jax
import jax.numpy as jnp
from jax import lax
from jax.experimental import pallas as pl
from jax.experimental.pallas import tpu as pltpu
from jax.experimental.pallas import tpu_sc as plsc

NCLS = 21
CPAD = 32
LANES = 16
BATCH = 8
HW = 512 * 512
TOTAL = BATCH * HW
NCORES = 2
NSUB = 16
NWORK = NCORES * NSUB
PER_W = TOTAL // NWORK
CHUNK = 16384
NCHUNK = PER_W // CHUNK
UNROLL = 8
HISTW = LANES * CPAD


def _dice_body(pred_hbm, label_hbm, out_hbm,
               pbuf0, lbuf0, pbuf1, lbuf1, hp, hl, hm, tmp, obuf,
               shared, shared2, sem_p, sem_l):
    c = lax.axis_index("c")
    s = lax.axis_index("s")
    w = c * NSUB + s
    zeros16 = jnp.zeros((LANES,), jnp.float32)
    ones16 = jnp.ones((LANES,), jnp.float32)
    lane = lax.iota(jnp.int32, LANES) * CPAD

    base = w * PER_W
    bufs = [(pbuf0, lbuf0), (pbuf1, lbuf1)]

    def start_chunk(ch):
        off = base + ch * CHUNK
        pb, lb = bufs[ch % 2]
        cp = pltpu.make_async_copy(pred_hbm.at[pl.ds(off, CHUNK)], pb, sem_p)
        cl = pltpu.make_async_copy(label_hbm.at[pl.ds(off, CHUNK)], lb, sem_l)
        cp.start()
        cl.start()
        return cp, cl

    pending = start_chunk(0)

    for i in range(CPAD):
        hp[pl.ds(i * LANES, LANES)] = zeros16
        hl[pl.ds(i * LANES, LANES)] = zeros16
        hm[pl.ds(i * LANES, LANES)] = zeros16

    for ch in range(NCHUNK):
        pending[0].wait()
        pending[1].wait()
        if ch + 1 < NCHUNK:
            pending = start_chunk(ch + 1)
        pb, lb = bufs[ch % 2]

        def body(j, carry):
            ibase = j * (LANES * UNROLL)
            for u in range(UNROLL):
                p16 = pb[pl.ds(ibase + u * LANES, LANES)]
                l16 = lb[pl.ds(ibase + u * LANES, LANES)]
                ip = lane + p16
                il = lane + l16
                plsc.addupdate_scatter(hp, [ip], ones16)
                plsc.addupdate_scatter(hl, [il], ones16)
                plsc.addupdate_scatter(hm, [ip], ones16, mask=p16 == l16)
            return carry

        lax.fori_loop(0, CHUNK // (LANES * UNROLL), body, 0)

    pltpu.sync_copy(hp, shared.at[pl.ds((s * 3 + 0) * HISTW, HISTW)])
    pltpu.sync_copy(hl, shared.at[pl.ds((s * 3 + 1) * HISTW, HISTW)])
    pltpu.sync_copy(hm, shared.at[pl.ds((s * 3 + 2) * HISTW, HISTW)])
    plsc.subcore_barrier()

    @pl.when(s < 4)
    def _():
        accs = []
        for h in range(3):
            a0 = zeros16
            a1 = zeros16
            for q in range(4):
                pltpu.sync_copy(
                    shared.at[pl.ds(((s * 4 + q) * 3 + h) * HISTW, HISTW)], tmp)
                for ln in range(LANES):
                    a0 = a0 + tmp[pl.ds(ln * CPAD, LANES)]
                    a1 = a1 + tmp[pl.ds(ln * CPAD + LANES, LANES)]
            accs.append((a0, a1))
        (p0, p1), (l0, l1), (m0, m1) = accs
        eps = jnp.float32(1e-10)
        inv_b = jnp.float32(1.0 / BATCH)
        s0 = (2.0 * m0) / (p0 + l0 + eps) * inv_b
        s1 = (2.0 * m1) / (p1 + l1 + eps) * inv_b
        obuf[pl.ds(0, LANES)] = s0
        obuf[pl.ds(LANES, LANES)] = s1
        pltpu.sync_copy(obuf, shared2.at[pl.ds(s * CPAD, CPAD)])

    plsc.subcore_barrier()

    @pl.when(s == 0)
    def _():
        t0 = zeros16
        t1 = zeros16
        for q in range(4):
            pltpu.sync_copy(shared2.at[pl.ds(q * CPAD, CPAD)], obuf)
            t0 = t0 + obuf[pl.ds(0, LANES)]
            t1 = t1 + obuf[pl.ds(LANES, LANES)]
        obuf[pl.ds(0, LANES)] = t0
        obuf[pl.ds(LANES, LANES)] = t1
        pltpu.sync_copy(obuf, out_hbm.at[pl.ds(c * CPAD, CPAD)])


@jax.jit
def _dice_call(pred_flat, label_flat):
    mesh = plsc.VectorSubcoreMesh(
        core_axis_name="c", subcore_axis_name="s",
        num_cores=NCORES, num_subcores=NSUB)
    return pl.kernel(
        _dice_body,
        out_type=jax.ShapeDtypeStruct((NCORES * CPAD,), jnp.float32),
        mesh=mesh,
        compiler_params=pltpu.CompilerParams(needs_layout_passes=False),
        scratch_types=[
            pltpu.VMEM((CHUNK,), jnp.int32),
            pltpu.VMEM((CHUNK,), jnp.int32),
            pltpu.VMEM((CHUNK,), jnp.int32),
            pltpu.VMEM((CHUNK,), jnp.int32),
            pltpu.VMEM((HISTW,), jnp.float32),
            pltpu.VMEM((HISTW,), jnp.float32),
            pltpu.VMEM((HISTW,), jnp.float32),
            pltpu.VMEM((HISTW,), jnp.float32),
            pltpu.VMEM((CPAD,), jnp.float32),
            pltpu.VMEM_SHARED((NSUB * 3 * HISTW,), jnp.float32),
            pltpu.VMEM_SHARED((4 * CPAD,), jnp.float32),
            pltpu.SemaphoreType.DMA,
            pltpu.SemaphoreType.DMA,
        ],
    )(pred_flat, label_flat)


def kernel(pred, label):
    parts = _dice_call(pred.reshape(TOTAL), label.reshape(TOTAL))
    return (parts[:NCLS] + parts[CPAD:CPAD + NCLS])


# speedup vs baseline: 1.0864x; 1.0864x over previous
import functools

import jax
import jax.numpy as jnp
from jax import lax
from jax.experimental import pallas as pl
from jax.experimental.pallas import tpu as pltpu
from jax.experimental.pallas import tpu_sc as plsc

NCLS = 21
CPAD = 32
LANES = 16
BATCH = 8
HW = 512 * 512
TOTAL = BATCH * HW
NCORES = 2
NSUB = 16
NWORK = NCORES * NSUB
PER_W = TOTAL // NWORK
CHUNK = 16384
NCHUNK = PER_W // CHUNK
UNROLL = 8
HISTW = NCLS * LANES


def _dice_body(pred_hbm, label_hbm, out_hbm,
               pbuf0, lbuf0, pbuf1, lbuf1, hp, hl, hm, tmp, cnt, obuf,
               shared, shared2, sem_p, sem_l):
    c = lax.axis_index("c")
    s = lax.axis_index("s")
    w = c * NSUB + s
    zeros16 = jnp.zeros((LANES,), jnp.float32)
    ones16 = jnp.ones((LANES,), jnp.float32)
    lane = lax.iota(jnp.int32, LANES)

    base = w * PER_W
    bufs = [(pbuf0, lbuf0), (pbuf1, lbuf1)]

    def start_chunk(ch):
        off = base + ch * CHUNK
        pb, lb = bufs[ch % 2]
        cp = pltpu.make_async_copy(pred_hbm.at[pl.ds(off, CHUNK)], pb, sem_p)
        cl = pltpu.make_async_copy(label_hbm.at[pl.ds(off, CHUNK)], lb, sem_l)
        cp.start()
        cl.start()
        return cp, cl

    pending = start_chunk(0)

    for i in range(NCLS):
        hp[pl.ds(i * LANES, LANES)] = zeros16
        hl[pl.ds(i * LANES, LANES)] = zeros16
        hm[pl.ds(i * LANES, LANES)] = zeros16

    for ch in range(NCHUNK):
        pending[0].wait()
        pending[1].wait()
        if ch + 1 < NCHUNK:
            pending = start_chunk(ch + 1)
        pb, lb = bufs[ch % 2]

        def body(j, carry):
            ibase = j * (LANES * UNROLL)
            for u in range(UNROLL):
                p16 = pb[pl.ds(ibase + u * LANES, LANES)]
                l16 = lb[pl.ds(ibase + u * LANES, LANES)]
                ip = p16 * LANES + lane
                il = l16 * LANES + lane
                plsc.addupdate_scatter(hp, [ip], ones16)
                plsc.addupdate_scatter(hl, [il], ones16)
                plsc.addupdate_scatter(hm, [ip], ones16, mask=p16 == l16)
            return carry

        lax.fori_loop(0, CHUNK // (LANES * UNROLL), body, 0)

    pltpu.sync_copy(hp, shared.at[pl.ds((s * 3 + 0) * HISTW, HISTW)])
    pltpu.sync_copy(hl, shared.at[pl.ds((s * 3 + 1) * HISTW, HISTW)])
    pltpu.sync_copy(hm, shared.at[pl.ds((s * 3 + 2) * HISTW, HISTW)])
    plsc.subcore_barrier()

    @pl.when(s < 4)
    def _():
        for i in range(6):
            cnt[pl.ds(i * LANES, LANES)] = zeros16
        last_lane = lane == (LANES - 1)
        for h in range(3):
            accs = [zeros16] * NCLS
            for q in range(4):
                pltpu.sync_copy(
                    shared.at[pl.ds(((s * 4 + q) * 3 + h) * HISTW, HISTW)], tmp)
                for cc in range(NCLS):
                    accs[cc] = accs[cc] + tmp[pl.ds(cc * LANES, LANES)]
            for cc in range(NCLS):
                tot = plsc.cumsum(accs[cc])
                plsc.store_scatter(
                    cnt, [jnp.full((LANES,), h * CPAD + cc, jnp.int32)],
                    tot, mask=last_lane)
        p0 = cnt[pl.ds(0, LANES)]
        p1 = cnt[pl.ds(LANES, LANES)]
        l0 = cnt[pl.ds(CPAD, LANES)]
        l1 = cnt[pl.ds(CPAD + LANES, LANES)]
        m0 = cnt[pl.ds(2 * CPAD, LANES)]
        m1 = cnt[pl.ds(2 * CPAD + LANES, LANES)]
        eps = jnp.float32(1e-10)
        inv_b = jnp.float32(1.0 / BATCH)
        s0 = (2.0 * m0) / (p0 + l0 + eps) * inv_b
        s1 = (2.0 * m1) / (p1 + l1 + eps) * inv_b
        obuf[pl.ds(0, LANES)] = s0
        obuf[pl.ds(LANES, LANES)] = s1
        pltpu.sync_copy(obuf, shared2.at[pl.ds(s * CPAD, CPAD)])

    plsc.subcore_barrier()

    @pl.when(s == 0)
    def _():
        t0 = zeros16
        t1 = zeros16
        for q in range(4):
            pltpu.sync_copy(shared2.at[pl.ds(q * CPAD, CPAD)], obuf)
            t0 = t0 + obuf[pl.ds(0, LANES)]
            t1 = t1 + obuf[pl.ds(LANES, LANES)]
        obuf[pl.ds(0, LANES)] = t0
        obuf[pl.ds(LANES, LANES)] = t1
        pltpu.sync_copy(obuf, out_hbm.at[pl.ds(c * CPAD, CPAD)])


@jax.jit
def _dice_call(pred_flat, label_flat):
    mesh = plsc.VectorSubcoreMesh(
        core_axis_name="c", subcore_axis_name="s",
        num_cores=NCORES, num_subcores=NSUB)
    return pl.kernel(
        _dice_body,
        out_type=jax.ShapeDtypeStruct((NCORES * CPAD,), jnp.float32),
        mesh=mesh,
        compiler_params=pltpu.CompilerParams(needs_layout_passes=False),
        scratch_types=[
            pltpu.VMEM((CHUNK,), jnp.int32),
            pltpu.VMEM((CHUNK,), jnp.int32),
            pltpu.VMEM((CHUNK,), jnp.int32),
            pltpu.VMEM((CHUNK,), jnp.int32),
            pltpu.VMEM((HISTW,), jnp.float32),
            pltpu.VMEM((HISTW,), jnp.float32),
            pltpu.VMEM((HISTW,), jnp.float32),
            pltpu.VMEM((HISTW,), jnp.float32),
            pltpu.VMEM((3 * CPAD,), jnp.float32),
            pltpu.VMEM((CPAD,), jnp.float32),
            pltpu.VMEM_SHARED((NSUB * 3 * HISTW,), jnp.float32),
            pltpu.VMEM_SHARED((4 * CPAD,), jnp.float32),
            pltpu.SemaphoreType.DMA,
            pltpu.SemaphoreType.DMA,
        ],
    )(pred_flat, label_flat)


def kernel(pred, label):
    parts = _dice_call(pred.reshape(TOTAL), label.reshape(TOTAL))
    return (parts[:NCLS] + parts[CPAD:CPAD + NCLS])


# speedup vs baseline: 1.0926x; 1.0057x over previous
import functools

import jax
import jax.numpy as jnp
from jax import lax
from jax.experimental import pallas as pl
from jax.experimental.pallas import tpu as pltpu
from jax.experimental.pallas import tpu_sc as plsc

NCLS = 21
CPAD = 32
LANES = 16
BATCH = 8
HW = 512 * 512
TOTAL = BATCH * HW
NCORES = 2
NSUB = 16
NWORK = NCORES * NSUB
PER_W = TOTAL // NWORK
CHUNK = 16384
NCHUNK = PER_W // CHUNK
UNROLL = 8
HISTW = NCLS * LANES


def _dice_body(pred_hbm, label_hbm, out_hbm,
               pbuf0, lbuf0, pbuf1, lbuf1, hpm_i, hl_i, hp, hl, hm, tmp, cnt,
               obuf, shared, shared2, sem_p, sem_l):
    c = lax.axis_index("c")
    s = lax.axis_index("s")
    w = c * NSUB + s
    zeros16 = jnp.zeros((LANES,), jnp.float32)
    zeros16i = jnp.zeros((LANES,), jnp.int32)
    ones16i = jnp.ones((LANES,), jnp.int32)
    lane = lax.iota(jnp.int32, LANES)

    base = w * PER_W
    bufs = [(pbuf0, lbuf0), (pbuf1, lbuf1)]

    def start_chunk(ch):
        off = base + ch * CHUNK
        pb, lb = bufs[ch % 2]
        cp = pltpu.make_async_copy(pred_hbm.at[pl.ds(off, CHUNK)], pb, sem_p)
        cl = pltpu.make_async_copy(label_hbm.at[pl.ds(off, CHUNK)], lb, sem_l)
        cp.start()
        cl.start()
        return cp, cl

    pending = start_chunk(0)

    for i in range(NCLS):
        hp[pl.ds(i * LANES, LANES)] = zeros16
        hl[pl.ds(i * LANES, LANES)] = zeros16
        hm[pl.ds(i * LANES, LANES)] = zeros16
        hpm_i[pl.ds(i * LANES, LANES)] = zeros16i
        hl_i[pl.ds(i * LANES, LANES)] = zeros16i

    for ch in range(NCHUNK):
        pending[0].wait()
        pending[1].wait()
        if ch + 1 < NCHUNK:
            pending = start_chunk(ch + 1)
        pb, lb = bufs[ch % 2]

        def body(j, carry):
            ibase = j * (LANES * UNROLL)
            for u in range(UNROLL):
                p16 = pb[pl.ds(ibase + u * LANES, LANES)]
                l16 = lb[pl.ds(ibase + u * LANES, LANES)]
                ip = p16 * LANES + lane
                il = l16 * LANES + lane
                eq = (p16 == l16).astype(jnp.int32)
                plsc.addupdate_scatter(hpm_i, [ip], ones16i + (eq << 15))
                plsc.addupdate_scatter(hl_i, [il], ones16i)
            return carry

        lax.fori_loop(0, CHUNK // (LANES * UNROLL), body, 0)

        for i in range(NCLS):
            sl = pl.ds(i * LANES, LANES)
            v = hpm_i[sl]
            hp[sl] = hp[sl] + (v & 0x7FFF).astype(jnp.float32)
            hm[sl] = hm[sl] + (v >> 15).astype(jnp.float32)
            hl[sl] = hl[sl] + hl_i[sl].astype(jnp.float32)
            hpm_i[sl] = zeros16i
            hl_i[sl] = zeros16i

    pltpu.sync_copy(hp, shared.at[pl.ds((s * 3 + 0) * HISTW, HISTW)])
    pltpu.sync_copy(hl, shared.at[pl.ds((s * 3 + 1) * HISTW, HISTW)])
    pltpu.sync_copy(hm, shared.at[pl.ds((s * 3 + 2) * HISTW, HISTW)])
    plsc.subcore_barrier()

    @pl.when(s < 4)
    def _():
        for i in range(6):
            cnt[pl.ds(i * LANES, LANES)] = zeros16
        last_lane = lane == (LANES - 1)
        for h in range(3):
            accs = [zeros16] * NCLS
            for q in range(4):
                pltpu.sync_copy(
                    shared.at[pl.ds(((s * 4 + q) * 3 + h) * HISTW, HISTW)], tmp)
                for cc in range(NCLS):
                    accs[cc] = accs[cc] + tmp[pl.ds(cc * LANES, LANES)]
            for cc in range(NCLS):
                tot = plsc.cumsum(accs[cc])
                plsc.store_scatter(
                    cnt, [jnp.full((LANES,), h * CPAD + cc, jnp.int32)],
                    tot, mask=last_lane)
        p0 = cnt[pl.ds(0, LANES)]
        p1 = cnt[pl.ds(LANES, LANES)]
        l0 = cnt[pl.ds(CPAD, LANES)]
        l1 = cnt[pl.ds(CPAD + LANES, LANES)]
        m0 = cnt[pl.ds(2 * CPAD, LANES)]
        m1 = cnt[pl.ds(2 * CPAD + LANES, LANES)]
        eps = jnp.float32(1e-10)
        inv_b = jnp.float32(1.0 / BATCH)
        s0 = (2.0 * m0) / (p0 + l0 + eps) * inv_b
        s1 = (2.0 * m1) / (p1 + l1 + eps) * inv_b
        obuf[pl.ds(0, LANES)] = s0
        obuf[pl.ds(LANES, LANES)] = s1
        pltpu.sync_copy(obuf, shared2.at[pl.ds(s * CPAD, CPAD)])

    plsc.subcore_barrier()

    @pl.when(s == 0)
    def _():
        t0 = zeros16
        t1 = zeros16
        for q in range(4):
            pltpu.sync_copy(shared2.at[pl.ds(q * CPAD, CPAD)], obuf)
            t0 = t0 + obuf[pl.ds(0, LANES)]
            t1 = t1 + obuf[pl.ds(LANES, LANES)]
        obuf[pl.ds(0, LANES)] = t0
        obuf[pl.ds(LANES, LANES)] = t1
        pltpu.sync_copy(obuf, out_hbm.at[pl.ds(c * CPAD, CPAD)])


@jax.jit
def _dice_call(pred_flat, label_flat):
    mesh = plsc.VectorSubcoreMesh(
        core_axis_name="c", subcore_axis_name="s",
        num_cores=NCORES, num_subcores=NSUB)
    return pl.kernel(
        _dice_body,
        out_type=jax.ShapeDtypeStruct((NCORES * CPAD,), jnp.float32),
        mesh=mesh,
        compiler_params=pltpu.CompilerParams(needs_layout_passes=False),
        scratch_types=[
            pltpu.VMEM((CHUNK,), jnp.int32),
            pltpu.VMEM((CHUNK,), jnp.int32),
            pltpu.VMEM((CHUNK,), jnp.int32),
            pltpu.VMEM((CHUNK,), jnp.int32),
            pltpu.VMEM((HISTW,), jnp.int32),
            pltpu.VMEM((HISTW,), jnp.int32),
            pltpu.VMEM((HISTW,), jnp.float32),
            pltpu.VMEM((HISTW,), jnp.float32),
            pltpu.VMEM((HISTW,), jnp.float32),
            pltpu.VMEM((HISTW,), jnp.float32),
            pltpu.VMEM((3 * CPAD,), jnp.float32),
            pltpu.VMEM((CPAD,), jnp.float32),
            pltpu.VMEM_SHARED((NSUB * 3 * HISTW,), jnp.float32),
            pltpu.VMEM_SHARED((4 * CPAD,), jnp.float32),
            pltpu.SemaphoreType.DMA,
            pltpu.SemaphoreType.DMA,
        ],
    )(pred_flat, label_flat)


def kernel(pred, label):
    parts = _dice_call(pred.reshape(TOTAL), label.reshape(TOTAL))
    return (parts[:NCLS] + parts[CPAD:CPAD + NCLS])


# speedup vs baseline: 1.4444x; 1.3220x over previous
import functools

import jax
import jax.numpy as jnp
from jax import lax
from jax.experimental import pallas as pl
from jax.experimental.pallas import tpu as pltpu
from jax.experimental.pallas import tpu_sc as plsc

NCLS = 21
CPAD = 32
LANES = 16
BATCH = 8
HW = 512 * 512
TOTAL = BATCH * HW
NCORES = 2
NSUB = 16
NWORK = NCORES * NSUB
PER_W = TOTAL // NWORK
ROWS_W = 512 // 4
CH_ROWS = 32
CHUNK = CH_ROWS * 512
NCHUNK = ROWS_W // CH_ROWS
HISTW = NCLS * LANES


def _dice_body(pred_hbm, label_hbm, out_hbm,
               pbuf0, lbuf0, pbuf1, lbuf1, hpm_i, hl_i, hp, hl, hm, tmp, cnt,
               obuf, shared, shared2, sem_p, sem_l):
    c = lax.axis_index("c")
    s = lax.axis_index("s")
    w = c * NSUB + s
    zeros16 = jnp.zeros((LANES,), jnp.float32)
    zeros16i = jnp.zeros((LANES,), jnp.int32)
    ones16i = jnp.ones((LANES,), jnp.int32)
    lane = lax.iota(jnp.int32, LANES)

    b = w // 4
    q = w % 4
    bufs = [(pbuf0, lbuf0), (pbuf1, lbuf1)]

    def start_chunk(ch):
        r0 = q * ROWS_W + ch * CH_ROWS
        pb, lb = bufs[ch % 2]
        cp = pltpu.make_async_copy(
            pred_hbm.at[pl.ds(b, 1), pl.ds(r0, CH_ROWS), :], pb, sem_p)
        cl = pltpu.make_async_copy(
            label_hbm.at[pl.ds(b, 1), pl.ds(r0, CH_ROWS), :], lb, sem_l)
        cp.start()
        cl.start()
        return cp, cl

    pending = start_chunk(0)

    for i in range(NCLS):
        hp[pl.ds(i * LANES, LANES)] = zeros16
        hl[pl.ds(i * LANES, LANES)] = zeros16
        hm[pl.ds(i * LANES, LANES)] = zeros16
        hpm_i[pl.ds(i * LANES, LANES)] = zeros16i
        hl_i[pl.ds(i * LANES, LANES)] = zeros16i

    for ch in range(NCHUNK):
        pending[0].wait()
        pending[1].wait()
        if ch + 1 < NCHUNK:
            pending = start_chunk(ch + 1)
        pb, lb = bufs[ch % 2]

        def body(r, carry):
            for k in range(512 // LANES):
                p16 = pb[0, r, pl.ds(k * LANES, LANES)]
                l16 = lb[0, r, pl.ds(k * LANES, LANES)]
                ip = p16 * LANES + lane
                il = l16 * LANES + lane
                eq = (p16 == l16).astype(jnp.int32)
                plsc.addupdate_scatter(hpm_i, [ip], ones16i + (eq << 15))
                plsc.addupdate_scatter(hl_i, [il], ones16i)
            return carry

        lax.fori_loop(0, CH_ROWS, body, 0)

        for i in range(NCLS):
            sl = pl.ds(i * LANES, LANES)
            v = hpm_i[sl]
            hp[sl] = hp[sl] + (v & 0x7FFF).astype(jnp.float32)
            hm[sl] = hm[sl] + (v >> 15).astype(jnp.float32)
            hl[sl] = hl[sl] + hl_i[sl].astype(jnp.float32)
            hpm_i[sl] = zeros16i
            hl_i[sl] = zeros16i

    pltpu.sync_copy(hp, shared.at[pl.ds((s * 3 + 0) * HISTW, HISTW)])
    pltpu.sync_copy(hl, shared.at[pl.ds((s * 3 + 1) * HISTW, HISTW)])
    pltpu.sync_copy(hm, shared.at[pl.ds((s * 3 + 2) * HISTW, HISTW)])
    plsc.subcore_barrier()

    @pl.when(s < 4)
    def _():
        for i in range(6):
            cnt[pl.ds(i * LANES, LANES)] = zeros16
        last_lane = lane == (LANES - 1)
        for h in range(3):
            accs = [zeros16] * NCLS
            for q in range(4):
                pltpu.sync_copy(
                    shared.at[pl.ds(((s * 4 + q) * 3 + h) * HISTW, HISTW)], tmp)
                for cc in range(NCLS):
                    accs[cc] = accs[cc] + tmp[pl.ds(cc * LANES, LANES)]
            for cc in range(NCLS):
                tot = plsc.cumsum(accs[cc])
                plsc.store_scatter(
                    cnt, [jnp.full((LANES,), h * CPAD + cc, jnp.int32)],
                    tot, mask=last_lane)
        p0 = cnt[pl.ds(0, LANES)]
        p1 = cnt[pl.ds(LANES, LANES)]
        l0 = cnt[pl.ds(CPAD, LANES)]
        l1 = cnt[pl.ds(CPAD + LANES, LANES)]
        m0 = cnt[pl.ds(2 * CPAD, LANES)]
        m1 = cnt[pl.ds(2 * CPAD + LANES, LANES)]
        eps = jnp.float32(1e-10)
        inv_b = jnp.float32(1.0 / BATCH)
        s0 = (2.0 * m0) / (p0 + l0 + eps) * inv_b
        s1 = (2.0 * m1) / (p1 + l1 + eps) * inv_b
        obuf[pl.ds(0, LANES)] = s0
        obuf[pl.ds(LANES, LANES)] = s1
        pltpu.sync_copy(obuf, shared2.at[pl.ds(s * CPAD, CPAD)])

    plsc.subcore_barrier()

    @pl.when(s == 0)
    def _():
        t0 = zeros16
        t1 = zeros16
        for q in range(4):
            pltpu.sync_copy(shared2.at[pl.ds(q * CPAD, CPAD)], obuf)
            t0 = t0 + obuf[pl.ds(0, LANES)]
            t1 = t1 + obuf[pl.ds(LANES, LANES)]
        obuf[pl.ds(0, LANES)] = t0
        obuf[pl.ds(LANES, LANES)] = t1
        pltpu.sync_copy(obuf, out_hbm.at[pl.ds(c * CPAD, CPAD)])


@jax.jit
def _dice_call(pred_flat, label_flat):
    mesh = plsc.VectorSubcoreMesh(
        core_axis_name="c", subcore_axis_name="s",
        num_cores=NCORES, num_subcores=NSUB)
    return pl.kernel(
        _dice_body,
        out_type=jax.ShapeDtypeStruct((NCORES * CPAD,), jnp.float32),
        mesh=mesh,
        compiler_params=pltpu.CompilerParams(
            needs_layout_passes=False, use_tc_tiling_on_sc=True),
        scratch_types=[
            pltpu.VMEM((1, CH_ROWS, 512), jnp.int32),
            pltpu.VMEM((1, CH_ROWS, 512), jnp.int32),
            pltpu.VMEM((1, CH_ROWS, 512), jnp.int32),
            pltpu.VMEM((1, CH_ROWS, 512), jnp.int32),
            pltpu.VMEM((HISTW,), jnp.int32),
            pltpu.VMEM((HISTW,), jnp.int32),
            pltpu.VMEM((HISTW,), jnp.float32),
            pltpu.VMEM((HISTW,), jnp.float32),
            pltpu.VMEM((HISTW,), jnp.float32),
            pltpu.VMEM((HISTW,), jnp.float32),
            pltpu.VMEM((3 * CPAD,), jnp.float32),
            pltpu.VMEM((CPAD,), jnp.float32),
            pltpu.VMEM_SHARED((NSUB * 3 * HISTW,), jnp.float32),
            pltpu.VMEM_SHARED((4 * CPAD,), jnp.float32),
            pltpu.SemaphoreType.DMA,
            pltpu.SemaphoreType.DMA,
        ],
    )(pred_flat, label_flat)


def kernel(pred, label):
    parts = _dice_call(pred.reshape(BATCH, 512, 512),
                       label.reshape(BATCH, 512, 512))
    return (parts[:NCLS] + parts[CPAD:CPAD + NCLS])
